# baseline (speedup 1.0000x reference)
import functools

import jax
import jax.numpy as jnp
from jax import lax
from jax.experimental import pallas as pl
from jax.experimental.pallas import tpu as pltpu
from jax.experimental.pallas import tpu_sc as plsc

_VOCAB = 100000
_EMB = 64
_CTX = 5
_B = 4096
_HID = 128
_FEAT = 2 * _CTX * _EMB

_VT = 1024
_NV = (_VOCAB + _VT - 1) // _VT

_NEG = -1e30


_SC_CORES = 2
_SC_SUBCORES = 16


def _build_sc_gather():
  nw = _SC_CORES * _SC_SUBCORES
  n_idx = _B * 2 * _CTX
  b_per_w = n_idx // nw
  mesh = plsc.VectorSubcoreMesh(core_axis_name="c", subcore_axis_name="s")

  @functools.partial(
      pl.kernel,
      mesh=mesh,
      compiler_params=pltpu.CompilerParams(use_tc_tiling_on_sc=False),
      out_type=jax.ShapeDtypeStruct((n_idx, _EMB), jnp.float32),
      scratch_types=[
          pltpu.VMEM((b_per_w,), jnp.int32),
          pltpu.VMEM((b_per_w, _EMB), jnp.float32),
          pltpu.SemaphoreType.DMA,
      ],
  )
  def gather_kernel(table_hbm, idx_hbm, out_hbm, idx_v, rows_v, sem):
    wid = lax.axis_index("s") * _SC_CORES + lax.axis_index("c")
    base = wid * b_per_w
    pltpu.sync_copy(idx_hbm.at[pl.ds(base, b_per_w)], idx_v)
    pltpu.async_copy(table_hbm.at[idx_v], rows_v, sem).wait()
    pltpu.sync_copy(rows_v, out_hbm.at[pl.ds(base, b_per_w)])

  return gather_kernel


_sc_gather_cache = []


def _sc_gather(table, idx):
  if not _sc_gather_cache:
    _sc_gather_cache.append(_build_sc_gather())
  return _sc_gather_cache[0](table, idx)


def _lse_body(embeds_ref, w1_ref, b1_ref, w2_ref, b2_ref,
              h_out, lse_out, m_ref, s_ref):
  j = pl.program_id(0)

  @pl.when(j == 0)
  def _init():
    h = jnp.dot(embeds_ref[...], w1_ref[...],
                preferred_element_type=jnp.float32)
    h_out[...] = jnp.maximum(h + b1_ref[...], 0.0)
    m_ref[...] = jnp.full_like(m_ref, _NEG)
    s_ref[...] = jnp.zeros_like(s_ref)

  logits = jnp.dot(h_out[...], w2_ref[...],
                   preferred_element_type=jnp.float32) + b2_ref[...]
  col = j * _VT + lax.broadcasted_iota(jnp.int32, logits.shape, 1)
  logits = jnp.where(col < _VOCAB, logits, _NEG)

  tmax = jnp.max(logits, axis=1, keepdims=True)
  m_old = m_ref[...]
  m_new = jnp.maximum(m_old, tmax)
  s_ref[...] = (s_ref[...] * jnp.exp(m_old - m_new)
                + jnp.sum(jnp.exp(logits - m_new), axis=1, keepdims=True))
  m_ref[...] = m_new

  @pl.when(j == _NV - 1)
  def _fini():
    lse_out[...] = m_ref[...] + jnp.log(s_ref[...])


def _run_lse(embeds, w1, b1_row, w2, b2_row):
  return pl.pallas_call(
      _lse_body,
      grid=(_NV,),
      in_specs=[
          pl.BlockSpec((_B, _FEAT), lambda j: (0, 0)),
          pl.BlockSpec((_FEAT, _HID), lambda j: (0, 0)),
          pl.BlockSpec((1, _HID), lambda j: (0, 0)),
          pl.BlockSpec((_HID, _VT), lambda j: (0, j)),
          pl.BlockSpec((1, _VT), lambda j: (0, j)),
      ],
      out_specs=[
          pl.BlockSpec((_B, _HID), lambda j: (0, 0)),
          pl.BlockSpec((_B, 1), lambda j: (0, 0)),
      ],
      out_shape=[
          jax.ShapeDtypeStruct((_B, _HID), jnp.float32),
          jax.ShapeDtypeStruct((_B, 1), jnp.float32),
      ],
      scratch_shapes=[
          pltpu.VMEM((_B, 1), jnp.float32),
          pltpu.VMEM((_B, 1), jnp.float32),
      ],
  )(embeds, w1, b1_row, w2, b2_row)


def _out_body(h_ref, lse_ref, w2_ref, b2_ref, o_ref):
  logits = jnp.dot(h_ref[...], w2_ref[...],
                   preferred_element_type=jnp.float32)
  o_ref[...] = logits + b2_ref[...] - lse_ref[...]


def _run_out(h, lse, w2, b2_row):
  return pl.pallas_call(
      _out_body,
      grid=(_NV,),
      in_specs=[
          pl.BlockSpec((_B, _HID), lambda j: (0, 0)),
          pl.BlockSpec((_B, 1), lambda j: (0, 0)),
          pl.BlockSpec((_HID, _VT), lambda j: (0, j)),
          pl.BlockSpec((1, _VT), lambda j: (0, j)),
      ],
      out_specs=pl.BlockSpec((_B, _VT), lambda j: (0, j)),
      out_shape=jax.ShapeDtypeStruct((_B, _VOCAB), jnp.float32),
  )(h, lse, w2, b2_row)


def kernel(inputs, emb, W1, b1, W2, b2):
  idx = inputs.reshape(-1)
  embeds = _sc_gather(emb, idx).reshape(_B, _FEAT)
  b1_row = b1.reshape(1, _HID)
  b2_row = b2.reshape(1, _VOCAB)
  h, lse = _run_lse(embeds, W1, b1_row, W2, b2_row)
  return _run_out(h, lse, W2, b2_row)


# speedup vs baseline: 1.0101x; 1.0101x over previous
import functools

import jax
import jax.numpy as jnp
from jax import lax
from jax.experimental import pallas as pl
from jax.experimental.pallas import tpu as pltpu
from jax.experimental.pallas import tpu_sc as plsc

_VOCAB = 100000
_EMB = 64
_CTX = 5
_B = 4096
_HID = 128
_FEAT = 2 * _CTX * _EMB

_VT = 1024
_NV = (_VOCAB + _VT - 1) // _VT

_NEG = -1e30


_SC_CORES = 2
_SC_SUBCORES = 16


def _build_sc_gather():
  nw = _SC_CORES * _SC_SUBCORES
  n_idx = _B * 2 * _CTX
  b_per_w = n_idx // nw
  mesh = plsc.VectorSubcoreMesh(core_axis_name="c", subcore_axis_name="s")

  @functools.partial(
      pl.kernel,
      mesh=mesh,
      compiler_params=pltpu.CompilerParams(use_tc_tiling_on_sc=False),
      out_type=jax.ShapeDtypeStruct((n_idx, _EMB), jnp.float32),
      scratch_types=[
          pltpu.VMEM((b_per_w,), jnp.int32),
          pltpu.VMEM((b_per_w, _EMB), jnp.float32),
          pltpu.SemaphoreType.DMA,
      ],
  )
  def gather_kernel(table_hbm, idx_hbm, out_hbm, idx_v, rows_v, sem):
    wid = lax.axis_index("s") * _SC_CORES + lax.axis_index("c")
    base = wid * b_per_w
    pltpu.sync_copy(idx_hbm.at[pl.ds(base, b_per_w)], idx_v)
    pltpu.async_copy(table_hbm.at[idx_v], rows_v, sem).wait()
    pltpu.sync_copy(rows_v, out_hbm.at[pl.ds(base, b_per_w)])

  return gather_kernel


_sc_gather_cache = []


def _sc_gather(table, idx):
  if not _sc_gather_cache:
    _sc_gather_cache.append(_build_sc_gather())
  return _sc_gather_cache[0](table, idx)


def _lse_body(embeds_ref, w1_ref, b1_ref, w2_ref, b2_ref,
              h_out, lse_out, m_ref, s_ref):
  j = pl.program_id(0)

  @pl.when(j == 0)
  def _init():
    h = jnp.dot(embeds_ref[...], w1_ref[...],
                preferred_element_type=jnp.float32)
    h_out[...] = jnp.maximum(h + b1_ref[...], 0.0).astype(jnp.bfloat16)
    m_ref[...] = jnp.full_like(m_ref, _NEG)
    s_ref[...] = jnp.zeros_like(s_ref)

  logits = jnp.dot(h_out[...], w2_ref[...],
                   preferred_element_type=jnp.float32) + b2_ref[...]
  col = j * _VT + lax.broadcasted_iota(jnp.int32, logits.shape, 1)
  logits = jnp.where(col < _VOCAB, logits, _NEG)

  tmax = jnp.max(logits, axis=1, keepdims=True)
  m_old = m_ref[...]
  m_new = jnp.maximum(m_old, tmax)
  s_ref[...] = (s_ref[...] * jnp.exp(m_old - m_new)
                + jnp.sum(jnp.exp(logits - m_new), axis=1, keepdims=True))
  m_ref[...] = m_new

  @pl.when(j == _NV - 1)
  def _fini():
    lse_out[...] = m_ref[...] + jnp.log(s_ref[...])


def _run_lse(embeds, w1, b1_row, w2, b2_row):
  return pl.pallas_call(
      _lse_body,
      grid=(_NV,),
      in_specs=[
          pl.BlockSpec((_B, _FEAT), lambda j: (0, 0)),
          pl.BlockSpec((_FEAT, _HID), lambda j: (0, 0)),
          pl.BlockSpec((1, _HID), lambda j: (0, 0)),
          pl.BlockSpec((_HID, _VT), lambda j: (0, j)),
          pl.BlockSpec((1, _VT), lambda j: (0, j)),
      ],
      out_specs=[
          pl.BlockSpec((_B, _HID), lambda j: (0, 0)),
          pl.BlockSpec((_B, 1), lambda j: (0, 0)),
      ],
      out_shape=[
          jax.ShapeDtypeStruct((_B, _HID), jnp.bfloat16),
          jax.ShapeDtypeStruct((_B, 1), jnp.float32),
      ],
      scratch_shapes=[
          pltpu.VMEM((_B, 1), jnp.float32),
          pltpu.VMEM((_B, 1), jnp.float32),
      ],
  )(embeds, w1, b1_row, w2, b2_row)


def _out_body(h_ref, lse_ref, w2_ref, b2_ref, o_ref):
  logits = jnp.dot(h_ref[...], w2_ref[...],
                   preferred_element_type=jnp.float32)
  o_ref[...] = logits + b2_ref[...] - lse_ref[...]


def _run_out(h, lse, w2, b2_row):
  return pl.pallas_call(
      _out_body,
      grid=(_NV,),
      in_specs=[
          pl.BlockSpec((_B, _HID), lambda j: (0, 0)),
          pl.BlockSpec((_B, 1), lambda j: (0, 0)),
          pl.BlockSpec((_HID, _VT), lambda j: (0, j)),
          pl.BlockSpec((1, _VT), lambda j: (0, j)),
      ],
      out_specs=pl.BlockSpec((_B, _VT), lambda j: (0, j)),
      out_shape=jax.ShapeDtypeStruct((_B, _VOCAB), jnp.float32),
  )(h, lse, w2, b2_row)


def kernel(inputs, emb, W1, b1, W2, b2):
  idx = inputs.reshape(-1)
  embeds = _sc_gather(emb, idx).reshape(_B, _FEAT)
  b1_row = b1.reshape(1, _HID)
  b2_row = b2.reshape(1, _VOCAB)
  w2b = W2.astype(jnp.bfloat16)
  h, lse = _run_lse(embeds, W1, b1_row, w2b, b2_row)
  return _run_out(h, lse, w2b, b2_row)
